# baseline (speedup 1.0000x reference)
import functools

import jax
import jax.numpy as jnp
from jax import lax
from jax.experimental import pallas as pl
from jax.experimental.pallas import tpu as pltpu
from jax.experimental.pallas import tpu_sc as plsc

NC = 2
NS = 16
NW = NC * NS

N_LINKS = 10000
H = 128
N_EDGES = 640000
N_PL = 800000
NUM_PATH_NODE = 200000


CH_PER_W = 160
E_CHUNKS = CH_PER_W * NW
T_ACC_ROWS = 10112
T_ROWS = 632

SEG = 11512
N_PASS = 9
NWIN = NC * N_PASS
NPN_PAD = SEG * NWIN
ACC_ROWS = 11520
ROWS_RD = 720
EB_PAD = 48


def _zero_buf(buf, cols):
    def body(i, _):
        for v in range(cols // 16):
            buf[i, pl.ds(v * 16, 16)] = jnp.zeros((16,), jnp.float32)
        return 0
    lax.fori_loop(0, 128, body, 0)


def _topo_body(x_hbm, src_hbm, dst_hbm, out_hbm,
               is_blk, id_blk, r0, r1, acc, sg0, sg1, ss0, ss1):
    c = lax.axis_index("c")
    s = lax.axis_index("s")
    w = c * NS + s
    base = s * T_ROWS

    _zero_buf(r0, H)
    for k in range(4):
        pltpu.sync_copy(r0, acc.at[pl.ds(base + k * 128, 128), :])
    pltpu.sync_copy(r0.at[pl.ds(0, T_ROWS - 512), :],
                    acc.at[pl.ds(base + 512, T_ROWS - 512), :])
    plsc.subcore_barrier()

    def block(jb, _):
        row0 = pl.multiple_of(w * CH_PER_W + jb * 8, 8)
        pltpu.sync_copy(src_hbm.at[pl.ds(row0, 8), :], is_blk)
        pltpu.sync_copy(dst_hbm.at[pl.ds(row0, 8), :], id_blk)
        for r in range(0, 8, 2):
            cg0 = pltpu.async_copy(x_hbm.at[is_blk.at[r]], r0, sg0)
            cg1 = pltpu.async_copy(x_hbm.at[is_blk.at[r + 1]], r1, sg1)
            cg0.wait()
            cs0 = pltpu.async_copy(r0, acc.at[id_blk.at[r]], ss0, add=True)
            cg1.wait()
            cs1 = pltpu.async_copy(r1, acc.at[id_blk.at[r + 1]], ss1, add=True)
            cs0.wait()
            cs1.wait()
        return 0
    lax.fori_loop(0, CH_PER_W // 8, block, 0)

    plsc.subcore_barrier()

    @pl.when(s < NS - 1)
    def _():
        pltpu.sync_copy(acc.at[pl.ds(base, T_ROWS), :],
                        out_hbm.at[c, pl.ds(base, T_ROWS), :])

    @pl.when(s == NS - 1)
    def _():
        last = N_LINKS - (NS - 1) * T_ROWS
        pltpu.sync_copy(acc.at[pl.ds((NS - 1) * T_ROWS, last), :],
                        out_hbm.at[c, pl.ds((NS - 1) * T_ROWS, last), :])


def _allo_body(y_hbm, path_hbm, link_hbm, eb_hbm, out_hbm,
               idx_p, idx_l, idx_p2, idx_l2, rows, zb, ebv, acc,
               sp, sl, sg, sp2, sl2, sg2, ss, ss2):
    c = lax.axis_index("c")
    s = lax.axis_index("s")

    pltpu.sync_copy(eb_hbm, ebv)
    i16 = lax.iota(jnp.int32, 16)

    def bound(g):
        return ebv[pl.ds(g, 16)][0]

    def transform(ipb, b, e_lo, e_hi, p0):
        for v in range(8):
            pv = ipb[pl.ds(v * 16, 16)]
            pos = (b + v * 16) + i16
            ok = (pos >= e_lo) & (pos < e_hi)
            ipb[pl.ds(v * 16, 16)] = jnp.where(ok, pv - p0, SEG)

    def pass_body(p, _):
        g = c * N_PASS + p
        e_lo = bound(g)
        e_hi = bound(g + 1)
        p0 = g * SEG

        _zero_buf(zb, H)
        for k in range(5):
            pltpu.sync_copy(zb, acc.at[pl.ds(s * ROWS_RD + k * 128, 128), :])
        pltpu.sync_copy(zb.at[pl.ds(0, ROWS_RD - 640), :],
                        acc.at[pl.ds(s * ROWS_RD + 640, ROWS_RD - 640), :])
        plsc.subcore_barrier()

        c0 = e_lo // 128
        nch = (e_hi - c0 * 128 + 127) // 128

        def chunk2(jj, _):
            j0 = jj * 32 + s
            j1 = j0 + 16
            b0 = (c0 + j0) * 128
            b1 = (c0 + j1) * 128
            cpp0 = pltpu.async_copy(path_hbm.at[pl.ds(b0, 128)], idx_p, sp)
            cpl0 = pltpu.async_copy(link_hbm.at[pl.ds(b0, 128)], idx_l, sl)
            cpp1 = pltpu.async_copy(path_hbm.at[pl.ds(b1, 128)], idx_p2, sp2)
            cpl1 = pltpu.async_copy(link_hbm.at[pl.ds(b1, 128)], idx_l2, sl2)
            cpl0.wait()
            cg0 = pltpu.async_copy(y_hbm.at[idx_l], rows, sg)
            cpl1.wait()
            cg1 = pltpu.async_copy(y_hbm.at[idx_l2], zb, sg2)
            cpp0.wait()
            transform(idx_p, b0, e_lo, e_hi, p0)
            cpp1.wait()
            transform(idx_p2, b1, e_lo, e_hi, p0)
            cg0.wait()
            cs0 = pltpu.async_copy(rows, acc.at[idx_p], ss, add=True)
            cg1.wait()
            cs1 = pltpu.async_copy(zb, acc.at[idx_p2], ss2, add=True)
            cs0.wait()
            cs1.wait()
            return 0

        def chunk1(j, _):
            b = (c0 + j) * 128
            cpp = pltpu.async_copy(path_hbm.at[pl.ds(b, 128)], idx_p, sp)
            cpl = pltpu.async_copy(link_hbm.at[pl.ds(b, 128)], idx_l, sl)
            cpl.wait()
            cpg = pltpu.async_copy(y_hbm.at[idx_l], rows, sg)
            cpp.wait()
            transform(idx_p, b, e_lo, e_hi, p0)
            cpg.wait()
            pltpu.sync_copy(rows, acc.at[idx_p], add=True)
            return 0

        nj = jnp.maximum((nch - s + 15) // 16, 0)
        lax.fori_loop(0, nj // 2, chunk2, 0)

        @pl.when(nj % 2 == 1)
        def _():
            chunk1((nj // 2) * 32 + s, 0)

        plsc.subcore_barrier()

        @pl.when(s < NS - 1)
        def _():
            pltpu.sync_copy(acc.at[pl.ds(s * ROWS_RD, ROWS_RD), :],
                            out_hbm.at[pl.ds(p0 + s * ROWS_RD, ROWS_RD), :])

        @pl.when(s == NS - 1)
        def _():
            last = SEG - (NS - 1) * ROWS_RD
            pltpu.sync_copy(acc.at[pl.ds((NS - 1) * ROWS_RD, last), :],
                            out_hbm.at[pl.ds(p0 + (NS - 1) * ROWS_RD, last), :])
        return 0

    lax.fori_loop(0, N_PASS, pass_body, 0)


def _embed_body(cap_ref, w_ref, b_ref, o_ref):
    o_ref[...] = jnp.tanh(cap_ref[...] * w_ref[...] + b_ref[...])


def _layer_body(agg_ref, x_ref, w_ref, u_ref, o_ref):
    a = agg_ref[0] + agg_ref[1]
    o_ref[...] = jnp.maximum(
        jnp.dot(a, w_ref[...], preferred_element_type=jnp.float32)
        + jnp.dot(x_ref[...], u_ref[...], preferred_element_type=jnp.float32),
        0.0,
    )


def _y_body(x_ref, wa_ref, y_ref):
    y_ref[...] = jnp.dot(x_ref[...], wa_ref[...],
                         preferred_element_type=jnp.float32)


def _head_body(p_ref, ba_ref, wo_ref, bo_ref, o_ref):
    h = jnp.maximum(p_ref[...] + ba_ref[...], 0.0)
    o_ref[...] = jnp.dot(h, wo_ref[...], preferred_element_type=jnp.float32) + bo_ref[...]


def _mean_body(x_ref, wm_ref, bm_ref, o_ref):
    o_ref[...] = (
        jnp.dot(x_ref[...], wm_ref[...], preferred_element_type=jnp.float32)
        + bm_ref[...]
    )


def kernel(capacity, link_edge_index, path_link_path, path_link_link,
           W_in, b_in, W_layers, U_layers, Wa, ba, Wo, bo, W_mean, b_mean):
    num_path = W_mean.shape[0]
    mesh = plsc.VectorSubcoreMesh(core_axis_name="c", subcore_axis_name="s")

    topo_sum = functools.partial(
        pl.kernel,
        out_type=jax.ShapeDtypeStruct((NC, N_LINKS, H), jnp.float32),
        mesh=mesh,
        scratch_types=[
            pltpu.VMEM((8, 128), jnp.int32),
            pltpu.VMEM((8, 128), jnp.int32),
            pltpu.VMEM((128, H), jnp.float32),
            pltpu.VMEM((128, H), jnp.float32),
            pltpu.VMEM_SHARED((T_ACC_ROWS, H), jnp.float32),
            pltpu.SemaphoreType.DMA,
            pltpu.SemaphoreType.DMA,
            pltpu.SemaphoreType.DMA,
            pltpu.SemaphoreType.DMA,
        ],
    )(_topo_body)

    allo_sum = functools.partial(
        pl.kernel,
        out_type=jax.ShapeDtypeStruct((NPN_PAD, H), jnp.float32),
        mesh=mesh,
        scratch_types=[
            pltpu.VMEM((128,), jnp.int32),
            pltpu.VMEM((128,), jnp.int32),
            pltpu.VMEM((128,), jnp.int32),
            pltpu.VMEM((128,), jnp.int32),
            pltpu.VMEM((128, H), jnp.float32),
            pltpu.VMEM((128, H), jnp.float32),
            pltpu.VMEM((EB_PAD,), jnp.int32),
            pltpu.VMEM_SHARED((ACC_ROWS, H), jnp.float32),
            pltpu.SemaphoreType.DMA,
            pltpu.SemaphoreType.DMA,
            pltpu.SemaphoreType.DMA,
            pltpu.SemaphoreType.DMA,
            pltpu.SemaphoreType.DMA,
            pltpu.SemaphoreType.DMA,
            pltpu.SemaphoreType.DMA,
            pltpu.SemaphoreType.DMA,
        ],
    )(_allo_body)

    x = pl.pallas_call(
        _embed_body,
        out_shape=jax.ShapeDtypeStruct((N_LINKS, H), jnp.float32),
    )(capacity, W_in, b_in.reshape(1, H))

    e_pad = E_CHUNKS * 128 - N_EDGES
    src1d = jnp.concatenate(
        [link_edge_index[0], jnp.zeros((e_pad,), jnp.int32)]
    ).reshape(E_CHUNKS, 128)
    dst1d = jnp.concatenate(
        [link_edge_index[1], jnp.full((e_pad,), N_LINKS, jnp.int32)]
    ).reshape(E_CHUNKS, 128)

    def layer_step(l, xc):
        w_l = lax.dynamic_index_in_dim(W_layers, l, keepdims=False)
        u_l = lax.dynamic_index_in_dim(U_layers, l, keepdims=False)
        agg2 = topo_sum(xc, src1d, dst1d)
        return pl.pallas_call(
            _layer_body,
            out_shape=jax.ShapeDtypeStruct((N_LINKS, H), jnp.float32),
        )(agg2, xc, w_l, u_l)

    fuzz = (link_edge_index[0, 0] >= jnp.int32(N_LINKS)).astype(jnp.int32)
    x = lax.fori_loop(0, W_layers.shape[0] + fuzz, layer_step, x)

    y = pl.pallas_call(
        _y_body,
        out_shape=jax.ShapeDtypeStruct((N_LINKS, H), jnp.float32),
    )(x, Wa)

    bounds = jnp.minimum(jnp.arange(NWIN + 1, dtype=jnp.int32) * SEG,
                         NUM_PATH_NODE)
    eb = jnp.searchsorted(path_link_path, bounds).astype(jnp.int32)
    eb = jnp.concatenate([eb, jnp.zeros((EB_PAD - NWIN - 1,), jnp.int32)])

    path_pad = jnp.concatenate(
        [path_link_path, jnp.zeros((128,), jnp.int32)])
    link_pad = jnp.concatenate(
        [path_link_link, jnp.zeros((128,), jnp.int32)])

    p_wa = allo_sum(y, path_pad, link_pad, eb)

    out = pl.pallas_call(
        _head_body,
        out_shape=jax.ShapeDtypeStruct((NUM_PATH_NODE, 1), jnp.float32),
        grid=(10,),
        in_specs=[
            pl.BlockSpec((NUM_PATH_NODE // 10, H), lambda i: (i, 0)),
            pl.BlockSpec((1, H), lambda i: (0, 0)),
            pl.BlockSpec((H, 1), lambda i: (0, 0)),
            pl.BlockSpec((1, 1), lambda i: (0, 0)),
        ],
        out_specs=pl.BlockSpec((NUM_PATH_NODE // 10, 1), lambda i: (i, 0)),
    )(p_wa, ba.reshape(1, H), Wo, bo.reshape(1, 1))

    xr = out.reshape(NUM_PATH_NODE // num_path, num_path)
    mean = pl.pallas_call(
        _mean_body,
        out_shape=jax.ShapeDtypeStruct((NUM_PATH_NODE // num_path, num_path),
                                       jnp.float32),
    )(xr, W_mean, b_mean.reshape(1, num_path))
    std = jnp.float32(1.0)
    return (mean, std)


# speedup vs baseline: 1.6048x; 1.6048x over previous
import functools

import jax
import jax.numpy as jnp
from jax import lax
from jax.experimental import pallas as pl
from jax.experimental.pallas import tpu as pltpu
from jax.experimental.pallas import tpu_sc as plsc

NC = 2
NS = 16
NW = NC * NS

N_LINKS = 10000
H = 128
N_EDGES = 640000
N_PL = 800000
NUM_PATH_NODE = 200000


CH_PER_W = 158
E_CHUNKS = CH_PER_W * NW
T_ACC_ROWS = 10112
T_ROWS = 632

SEG = 11512
N_PASS = 9
NWIN = NC * N_PASS
NPN_PAD = SEG * NWIN
ACC_ROWS = 11520
ROWS_RD = 720
EB_PAD = 48


def _zero_buf(buf, cols):
    def body(i, _):
        for v in range(cols // 16):
            buf[i, pl.ds(v * 16, 16)] = jnp.zeros((16,), jnp.float32)
        return 0
    lax.fori_loop(0, 128, body, 0)


def _topo_body(x_hbm, src_hbm, dst_hbm, out_hbm,
               is0, is1, id0, id1, r0, r1, acc, sg0, sg1):
    c = lax.axis_index("c")
    s = lax.axis_index("s")
    w = c * NS + s
    base = s * T_ROWS

    _zero_buf(r0, H)
    for k in range(4):
        pltpu.sync_copy(r0, acc.at[pl.ds(base + k * 128, 128), :])
    pltpu.sync_copy(r0.at[pl.ds(0, T_ROWS - 512), :],
                    acc.at[pl.ds(base + 512, T_ROWS - 512), :])
    plsc.subcore_barrier()

    iss = (is0, is1)
    ids = (id0, id1)
    rows = (r0, r1)
    sgs = (sg0, sg1)

    def pair(jj, _):
        cps = []
        for b in range(2):
            eb0 = (w * CH_PER_W + jj * 2 + b) * 128
            pltpu.sync_copy(src_hbm.at[pl.ds(eb0, 128)], iss[b])
            pltpu.sync_copy(dst_hbm.at[pl.ds(eb0, 128)], ids[b])
            cps.append(pltpu.async_copy(x_hbm.at[iss[b]], rows[b], sgs[b]))
        for b in range(2):
            cps[b].wait()
            pltpu.sync_copy(rows[b], acc.at[ids[b]], add=True)
        return 0
    lax.fori_loop(0, CH_PER_W // 2, pair, 0)

    plsc.subcore_barrier()

    @pl.when(s < NS - 1)
    def _():
        pltpu.sync_copy(acc.at[pl.ds(base, T_ROWS), :],
                        out_hbm.at[c, pl.ds(base, T_ROWS), :])

    @pl.when(s == NS - 1)
    def _():
        last = N_LINKS - (NS - 1) * T_ROWS
        pltpu.sync_copy(acc.at[pl.ds((NS - 1) * T_ROWS, last), :],
                        out_hbm.at[c, pl.ds((NS - 1) * T_ROWS, last), :])


def _allo_body(y_hbm, path_hbm, link_hbm, eb_hbm, out_hbm,
               idx_p, idx_l, rows, zb, ebv, acc, sp, sl, sg):
    c = lax.axis_index("c")
    s = lax.axis_index("s")

    _zero_buf(zb, H)
    pltpu.sync_copy(eb_hbm, ebv)
    i16 = lax.iota(jnp.int32, 16)

    def bound(g):
        return ebv[pl.ds(g, 16)][0]

    def pass_body(p, _):
        g = c * N_PASS + p
        e_lo = bound(g)
        e_hi = bound(g + 1)
        p0 = g * SEG

        for k in range(5):
            pltpu.sync_copy(zb, acc.at[pl.ds(s * ROWS_RD + k * 128, 128), :])
        pltpu.sync_copy(zb.at[pl.ds(0, ROWS_RD - 640), :],
                        acc.at[pl.ds(s * ROWS_RD + 640, ROWS_RD - 640), :])
        plsc.subcore_barrier()

        c0 = e_lo // 128
        nch = (e_hi - c0 * 128 + 127) // 128

        def chunk(jj, _):
            j = jj * 16 + s
            b = (c0 + j) * 128
            cpp = pltpu.async_copy(path_hbm.at[pl.ds(b, 128)], idx_p, sp)
            cpl = pltpu.async_copy(link_hbm.at[pl.ds(b, 128)], idx_l, sl)
            cpl.wait()
            cpg = pltpu.async_copy(y_hbm.at[idx_l], rows, sg)
            cpp.wait()
            for v in range(8):
                pv = idx_p[pl.ds(v * 16, 16)]
                pos = (b + v * 16) + i16
                ok = (pos >= e_lo) & (pos < e_hi)
                idx_p[pl.ds(v * 16, 16)] = jnp.where(ok, pv - p0, SEG)
            cpg.wait()
            pltpu.sync_copy(rows, acc.at[idx_p], add=True)
            return 0

        nj = jnp.maximum((nch - s + 15) // 16, 0)
        lax.fori_loop(0, nj, chunk, 0)
        plsc.subcore_barrier()

        @pl.when(s < NS - 1)
        def _():
            pltpu.sync_copy(acc.at[pl.ds(s * ROWS_RD, ROWS_RD), :],
                            out_hbm.at[pl.ds(p0 + s * ROWS_RD, ROWS_RD), :])

        @pl.when(s == NS - 1)
        def _():
            last = SEG - (NS - 1) * ROWS_RD
            pltpu.sync_copy(acc.at[pl.ds((NS - 1) * ROWS_RD, last), :],
                            out_hbm.at[pl.ds(p0 + (NS - 1) * ROWS_RD, last), :])
        return 0

    lax.fori_loop(0, N_PASS, pass_body, 0)


def _embed_body(cap_ref, w_ref, b_ref, o_ref):
    o_ref[...] = jnp.tanh(cap_ref[...] * w_ref[...] + b_ref[...])


def _layer_body(agg_ref, x_ref, w_ref, u_ref, o_ref):
    a = agg_ref[0] + agg_ref[1]
    o_ref[...] = jnp.maximum(
        jnp.dot(a, w_ref[...], preferred_element_type=jnp.float32)
        + jnp.dot(x_ref[...], u_ref[...], preferred_element_type=jnp.float32),
        0.0,
    )


def _y_body(x_ref, wa_ref, y_ref):
    y_ref[...] = jnp.dot(x_ref[...], wa_ref[...],
                         preferred_element_type=jnp.float32)


def _head_body(p_ref, ba_ref, wo_ref, bo_ref, o_ref):
    h = jnp.maximum(p_ref[...] + ba_ref[...], 0.0)
    o_ref[...] = jnp.dot(h, wo_ref[...], preferred_element_type=jnp.float32) + bo_ref[...]


def _mean_body(x_ref, wm_ref, bm_ref, o_ref):
    o_ref[...] = (
        jnp.dot(x_ref[...], wm_ref[...], preferred_element_type=jnp.float32)
        + bm_ref[...]
    )


def kernel(capacity, link_edge_index, path_link_path, path_link_link,
           W_in, b_in, W_layers, U_layers, Wa, ba, Wo, bo, W_mean, b_mean):
    num_path = W_mean.shape[0]
    mesh = plsc.VectorSubcoreMesh(core_axis_name="c", subcore_axis_name="s")

    topo_sum = functools.partial(
        pl.kernel,
        out_type=jax.ShapeDtypeStruct((NC, N_LINKS, H), jnp.float32),
        mesh=mesh,
        scratch_types=[
            pltpu.VMEM((128,), jnp.int32),
            pltpu.VMEM((128,), jnp.int32),
            pltpu.VMEM((128,), jnp.int32),
            pltpu.VMEM((128,), jnp.int32),
            pltpu.VMEM((128, H), jnp.float32),
            pltpu.VMEM((128, H), jnp.float32),
            pltpu.VMEM_SHARED((T_ACC_ROWS, H), jnp.float32),
            pltpu.SemaphoreType.DMA,
            pltpu.SemaphoreType.DMA,
        ],
    )(_topo_body)

    allo_sum = functools.partial(
        pl.kernel,
        out_type=jax.ShapeDtypeStruct((NPN_PAD, H), jnp.float32),
        mesh=mesh,
        scratch_types=[
            pltpu.VMEM((128,), jnp.int32),
            pltpu.VMEM((128,), jnp.int32),
            pltpu.VMEM((128, H), jnp.float32),
            pltpu.VMEM((128, H), jnp.float32),
            pltpu.VMEM((EB_PAD,), jnp.int32),
            pltpu.VMEM_SHARED((ACC_ROWS, H), jnp.float32),
            pltpu.SemaphoreType.DMA,
            pltpu.SemaphoreType.DMA,
            pltpu.SemaphoreType.DMA,
        ],
    )(_allo_body)

    x = pl.pallas_call(
        _embed_body,
        out_shape=jax.ShapeDtypeStruct((N_LINKS, H), jnp.float32),
    )(capacity, W_in, b_in.reshape(1, H))

    e_pad = E_CHUNKS * 128 - N_EDGES
    src1d = jnp.concatenate(
        [link_edge_index[0], jnp.zeros((e_pad,), jnp.int32)])
    dst1d = jnp.concatenate(
        [link_edge_index[1], jnp.full((e_pad,), N_LINKS, jnp.int32)])

    def layer_step(l, xc):
        w_l = lax.dynamic_index_in_dim(W_layers, l, keepdims=False)
        u_l = lax.dynamic_index_in_dim(U_layers, l, keepdims=False)
        agg2 = topo_sum(xc, src1d, dst1d)
        return pl.pallas_call(
            _layer_body,
            out_shape=jax.ShapeDtypeStruct((N_LINKS, H), jnp.float32),
        )(agg2, xc, w_l, u_l)

    fuzz = (link_edge_index[0, 0] >= jnp.int32(N_LINKS)).astype(jnp.int32)
    x = lax.fori_loop(0, W_layers.shape[0] + fuzz, layer_step, x)

    y = pl.pallas_call(
        _y_body,
        out_shape=jax.ShapeDtypeStruct((N_LINKS, H), jnp.float32),
    )(x, Wa)

    bounds = jnp.minimum(jnp.arange(NWIN + 1, dtype=jnp.int32) * SEG,
                         NUM_PATH_NODE)
    eb = jnp.searchsorted(path_link_path, bounds).astype(jnp.int32)
    eb = jnp.concatenate([eb, jnp.zeros((EB_PAD - NWIN - 1,), jnp.int32)])

    path_pad = jnp.concatenate(
        [path_link_path, jnp.zeros((128,), jnp.int32)])
    link_pad = jnp.concatenate(
        [path_link_link, jnp.zeros((128,), jnp.int32)])

    p_wa = allo_sum(y, path_pad, link_pad, eb)

    out = pl.pallas_call(
        _head_body,
        out_shape=jax.ShapeDtypeStruct((NUM_PATH_NODE, 1), jnp.float32),
        grid=(10,),
        in_specs=[
            pl.BlockSpec((NUM_PATH_NODE // 10, H), lambda i: (i, 0)),
            pl.BlockSpec((1, H), lambda i: (0, 0)),
            pl.BlockSpec((H, 1), lambda i: (0, 0)),
            pl.BlockSpec((1, 1), lambda i: (0, 0)),
        ],
        out_specs=pl.BlockSpec((NUM_PATH_NODE // 10, 1), lambda i: (i, 0)),
    )(p_wa, ba.reshape(1, H), Wo, bo.reshape(1, 1))

    xr = out.reshape(NUM_PATH_NODE // num_path, num_path)
    mean = pl.pallas_call(
        _mean_body,
        out_shape=jax.ShapeDtypeStruct((NUM_PATH_NODE // num_path, num_path),
                                       jnp.float32),
    )(xr, W_mean, b_mean.reshape(1, num_path))
    std = jnp.float32(1.0)
    return (mean, std)
